# baseline (speedup 1.0000x reference)
import functools

import jax
import jax.numpy as jnp
from jax import lax
from jax.experimental import pallas as pl
from jax.experimental.pallas import tpu as pltpu
from jax.experimental.pallas import tpu_sc as plsc

VOCAB = 49408
SEQ = 77
D = 768
BATCH = 4096
N = BATCH * SEQ
NC = 2
NS = 16
NW = NC * NS
RPW = N // NW
C = 11
NCH = RPW // C
LANES = 16
DV = D // LANES


def _emb_kernel(idx_hbm, tok_hbm, pos_hbm, out_hbm, idx_v, pos_v, buf, sem):
    wid = lax.axis_index("s") * NC + lax.axis_index("c")
    wbase = wid * RPW

    pltpu.sync_copy(idx_hbm.at[pl.ds(wid * NCH, NCH)], idx_v)
    pltpu.sync_copy(pos_hbm, pos_v)

    @pl.loop(0, NCH)
    def chunk_body(g):
        pltpu.async_copy(tok_hbm.at[idx_v.at[g]], buf, sem).wait()
        pb = lax.rem(g, 7) * C
        for r in range(C):
            for d in range(DV):
                plsc.addupdate(
                    buf.at[r, pl.ds(d * LANES, LANES)],
                    pos_v[pb + r, pl.ds(d * LANES, LANES)],
                )
        pltpu.async_copy(buf, out_hbm.at[pl.ds(wbase + g * C, C)], sem).wait()


@jax.jit
def _emb(idx2d, token_table, pos_table):
    mesh = plsc.VectorSubcoreMesh(
        core_axis_name="c", subcore_axis_name="s", num_cores=NC, num_subcores=NS
    )
    f = functools.partial(
        pl.kernel,
        out_type=jax.ShapeDtypeStruct((N, D), jnp.float32),
        mesh=mesh,
        scratch_types=[
            pltpu.VMEM((NCH, C), jnp.int32),
            pltpu.VMEM((SEQ, D), jnp.float32),
            pltpu.VMEM((C, D), jnp.float32),
            pltpu.SemaphoreType.DMA,
        ],
        compiler_params=pltpu.CompilerParams(use_tc_tiling_on_sc=False),
    )(_emb_kernel)
    return f(idx2d, token_table, pos_table)


def kernel(x, token_table, pos_table):
    idx2d = x.astype(jnp.int32).reshape(NW * NCH, C)
    out = _emb(idx2d, token_table, pos_table)
    return out.reshape(BATCH, SEQ, D)


# speedup vs baseline: 2.2657x; 2.2657x over previous
import functools

import jax
import jax.numpy as jnp
from jax import lax
from jax.experimental import pallas as pl
from jax.experimental.pallas import tpu as pltpu
from jax.experimental.pallas import tpu_sc as plsc

VOCAB = 49408
SEQ = 77
D = 768
BATCH = 4096
N = BATCH * SEQ
NC = 2
NS = 16
NW = NC * NS
RPW = N // NW
C = 11
NBUF = 7
SPW = RPW // SEQ
NCH = RPW // C
LEAD = 5
LANES = 16
DV = D // LANES


def _gather(tok_hbm, idx_ref, buf, gsem, slot):
    return pltpu.make_async_copy(tok_hbm.at[idx_ref], buf.at[slot],
                                 gsem.at[slot])


def _scatter(out_hbm, buf, ssem, wbase, g, slot):
    return pltpu.make_async_copy(buf.at[slot],
                                 out_hbm.at[pl.ds(wbase + g * C, C)],
                                 ssem.at[slot])


def _emb_kernel(idx_hbm, tok_hbm, pos_hbm, out_hbm, idx_v, pos_v, buf,
                gsem, ssem, isem):
    wid = lax.axis_index("s") * NC + lax.axis_index("c")
    wbase = wid * RPW

    pltpu.sync_copy(pos_hbm, pos_v)
    pltpu.sync_copy(idx_hbm.at[pl.ds(wid * NCH, NBUF)], idx_v.at[0])

    for b in range(LEAD):
        _gather(tok_hbm, idx_v.at[0, b], buf, gsem, b).start()

    @pl.loop(0, SPW)
    def seq_body(i):
        g0 = i * NBUF
        sl = lax.rem(i, 2)
        sl2 = lax.rem(i + 1, 2)

        @pl.when(i + 1 < SPW)
        def _():
            pltpu.make_async_copy(
                idx_hbm.at[pl.ds(wid * NCH + (i + 1) * NBUF, NBUF)],
                idx_v.at[sl2], isem).start()

        for b in range(NBUF):
            g = g0 + b
            _gather(tok_hbm, idx_v.at[sl, b], buf, gsem, b).wait()
            pb = b * C

            @pl.loop(0, C)
            def row_body(r):
                for d in range(DV):
                    plsc.addupdate(
                        buf.at[b, r, pl.ds(d * LANES, LANES)],
                        pos_v[pb + r, pl.ds(d * LANES, LANES)],
                    )
            _scatter(out_hbm, buf, ssem, wbase, g, b).start()

            if b == 2:
                @pl.when(i + 1 < SPW)
                def _():
                    pltpu.make_async_copy(
                        idx_hbm.at[pl.ds(wid * NCH + (i + 1) * NBUF, NBUF)],
                        idx_v.at[sl2], isem).wait()

            slot2 = (b + LEAD) % NBUF
            g2 = g + LEAD
            b2 = (b + LEAD) % NBUF
            isl = sl if b + LEAD < NBUF else sl2

            @pl.when(g2 < NCH)
            def _():
                @pl.when(g2 >= NBUF)
                def _():
                    _scatter(out_hbm, buf, ssem, wbase, g2 - NBUF,
                             slot2).wait()
                _gather(tok_hbm, idx_v.at[isl, b2], buf, gsem,
                        slot2).start()

    for b in range(NBUF):
        _scatter(out_hbm, buf, ssem, wbase, NCH - NBUF + b, b).wait()


@jax.jit
def _emb(idx2d, token_table, pos_table):
    mesh = plsc.VectorSubcoreMesh(
        core_axis_name="c", subcore_axis_name="s", num_cores=NC, num_subcores=NS
    )
    f = functools.partial(
        pl.kernel,
        out_type=jax.ShapeDtypeStruct((N, D), jnp.float32),
        mesh=mesh,
        scratch_types=[
            pltpu.VMEM((2, NBUF, C), jnp.int32),
            pltpu.VMEM((SEQ, D), jnp.float32),
            pltpu.VMEM((NBUF, C, D), jnp.float32),
            pltpu.SemaphoreType.DMA((NBUF,)),
            pltpu.SemaphoreType.DMA((NBUF,)),
            pltpu.SemaphoreType.DMA,
        ],
        compiler_params=pltpu.CompilerParams(use_tc_tiling_on_sc=False),
    )(_emb_kernel)
    return f(idx2d, token_table, pos_table)


def kernel(x, token_table, pos_table):
    idx2d = x.astype(jnp.int32).reshape(NW * NCH, C)
    out = _emb(idx2d, token_table, pos_table)
    return out.reshape(BATCH, SEQ, D)


# speedup vs baseline: 2.2705x; 1.0021x over previous
import functools

import jax
import jax.numpy as jnp
from jax import lax
from jax.experimental import pallas as pl
from jax.experimental.pallas import tpu as pltpu
from jax.experimental.pallas import tpu_sc as plsc

VOCAB = 49408
SEQ = 77
D = 768
BATCH = 4096
N = BATCH * SEQ
NC = 2
NS = 16
NW = NC * NS
RPW = N // NW
C = 11
NBUF = 7
SPW = RPW // SEQ
NCH = RPW // C
LEAD = 5
LANES = 16
DV = D // LANES


def _gather(tok_hbm, idx_ref, buf, gsem, slot):
    return pltpu.make_async_copy(tok_hbm.at[idx_ref], buf.at[slot],
                                 gsem.at[slot])


def _scatter(out_hbm, buf, ssem, wbase, g, slot):
    return pltpu.make_async_copy(buf.at[slot],
                                 out_hbm.at[pl.ds(wbase + g * C, C)],
                                 ssem.at[slot])


def _emb_kernel(idx_hbm, tok_hbm, pos_hbm, out_hbm, idx_v, pos_v, buf,
                gsem, ssem, isem):
    wid = lax.axis_index("s") * NC + lax.axis_index("c")
    wbase = wid * RPW

    pltpu.sync_copy(pos_hbm, pos_v)
    pltpu.sync_copy(idx_hbm.at[pl.ds(wid * NCH, NBUF)], idx_v.at[0])

    for b in range(LEAD):
        _gather(tok_hbm, idx_v.at[0, b], buf, gsem, b).start()

    @pl.loop(0, SPW)
    def seq_body(i):
        g0 = i * NBUF
        sl = lax.rem(i, 2)
        sl2 = lax.rem(i + 1, 2)

        @pl.when(i + 1 < SPW)
        def _():
            pltpu.make_async_copy(
                idx_hbm.at[pl.ds(wid * NCH + (i + 1) * NBUF, NBUF)],
                idx_v.at[sl2], isem).start()

        for b in range(NBUF):
            g = g0 + b
            _gather(tok_hbm, idx_v.at[sl, b], buf, gsem, b).wait()
            pb = b * C

            @plsc.parallel_loop(0, C)
            def row_body(r):
                for g0 in range(0, DV, 6):
                    vals = [
                        pos_v[pb + r, pl.ds((g0 + j) * LANES, LANES)]
                        for j in range(6)
                    ]
                    for j in range(6):
                        plsc.addupdate(
                            buf.at[b, r, pl.ds((g0 + j) * LANES, LANES)],
                            vals[j],
                        )
            _scatter(out_hbm, buf, ssem, wbase, g, b).start()

            if b == 2:
                @pl.when(i + 1 < SPW)
                def _():
                    pltpu.make_async_copy(
                        idx_hbm.at[pl.ds(wid * NCH + (i + 1) * NBUF, NBUF)],
                        idx_v.at[sl2], isem).wait()

            slot2 = (b + LEAD) % NBUF
            g2 = g + LEAD
            b2 = (b + LEAD) % NBUF
            isl = sl if b + LEAD < NBUF else sl2

            @pl.when(g2 < NCH)
            def _():
                @pl.when(g2 >= NBUF)
                def _():
                    _scatter(out_hbm, buf, ssem, wbase, g2 - NBUF,
                             slot2).wait()
                _gather(tok_hbm, idx_v.at[isl, b2], buf, gsem,
                        slot2).start()

    for b in range(NBUF):
        _scatter(out_hbm, buf, ssem, wbase, NCH - NBUF + b, b).wait()


@jax.jit
def _emb(idx2d, token_table, pos_table):
    mesh = plsc.VectorSubcoreMesh(
        core_axis_name="c", subcore_axis_name="s", num_cores=NC, num_subcores=NS
    )
    f = functools.partial(
        pl.kernel,
        out_type=jax.ShapeDtypeStruct((N, D), jnp.float32),
        mesh=mesh,
        scratch_types=[
            pltpu.VMEM((2, NBUF, C), jnp.int32),
            pltpu.VMEM((SEQ, D), jnp.float32),
            pltpu.VMEM((NBUF, C, D), jnp.float32),
            pltpu.SemaphoreType.DMA((NBUF,)),
            pltpu.SemaphoreType.DMA((NBUF,)),
            pltpu.SemaphoreType.DMA,
        ],
        compiler_params=pltpu.CompilerParams(use_tc_tiling_on_sc=False),
    )(_emb_kernel)
    return f(idx2d, token_table, pos_table)


def kernel(x, token_table, pos_table):
    idx2d = x.astype(jnp.int32).reshape(NW * NCH, C)
    out = _emb(idx2d, token_table, pos_table)
    return out.reshape(BATCH, SEQ, D)
